# baseline (speedup 1.0000x reference)
import functools

import jax
import jax.numpy as jnp
from jax import lax
from jax.experimental import pallas as pl
from jax.experimental.pallas import tpu as pltpu
from jax.experimental.pallas import tpu_sc as plsc

N = 10000
D = 128
E = 320000
BN_EPS = 1e-5

NC = 2
NS = 16
NW = NC * NS
CHUNK = 64
GPT = 160
PHASE = 40
PAIRC = NC * GPT
GF = 280
GS = PAIRC - GF
FAST_CORE = 0
EPAD = NW * GPT * CHUNK
NACC = 10240
ZROWS = NACC // NS
OROWS = NACC // NS

def _sc_agg_body(h_hbm, src_hbm, dst_hbm, zeros_hbm, out_hbm,
                 src_v, dst_v, rows0, rows1, rows2, rows3, acc,
                 gsem0, gsem1, gsem2, gsem3, ssem0, ssem1, ssem2, ssem3):
    c = lax.axis_index("c")
    s = lax.axis_index("s")
    w = s * NC + c

    pltpu.sync_copy(zeros_hbm.at[pl.ds(s * ZROWS, ZROWS)],
                    acc.at[pl.ds(s * ZROWS, ZROWS)])
    plsc.subcore_barrier()

    def g_issue(buf, sem, chunk):
        pltpu.async_copy(h_hbm.at[src_v.at[chunk]], buf, sem)

    def g_wait(buf, sem):
        pltpu.make_async_copy(h_hbm.at[src_v.at[0]], buf, sem).wait()

    def s_issue(buf, sem, chunk):
        pltpu.async_copy(buf, acc.at[dst_v.at[chunk]], sem, add=True)

    def s_wait(buf, sem):
        pltpu.make_async_copy(buf, acc.at[dst_v.at[0]], sem).wait()

    bufs = ((rows0, gsem0, ssem0), (rows1, gsem1, ssem1),
            (rows2, gsem2, ssem2), (rows3, gsem3, ssem3))

    def phase(off):
        pltpu.sync_copy(src_hbm.at[pl.ds(off, PHASE)], src_v)
        pltpu.sync_copy(dst_hbm.at[pl.ds(off, PHASE)], dst_v)

        g_issue(rows0, gsem0, 0)
        g_issue(rows1, gsem1, 1)
        g_issue(rows2, gsem2, 2)
        g_wait(rows0, gsem0)
        s_issue(rows0, ssem0, 0)
        g_issue(rows3, gsem3, 3)

        def body(j, carry):
            a = 4 * j
            for k in range(4):
                buf, gs, ss = bufs[(k + 1) % 4]
                pbuf, pgs, pss = bufs[k]
                g_wait(buf, gs)
                s_wait(pbuf, pss)
                s_issue(buf, ss, a + k + 1)
                g_issue(pbuf, pgs, a + k + 4)
            return carry

        lax.fori_loop(0, (PHASE - 8) // 4 + 1, body, 0)
        g_wait(rows1, gsem1)
        s_wait(rows0, ssem0)
        s_issue(rows1, ssem1, PHASE - 3)
        g_wait(rows2, gsem2)
        s_wait(rows1, ssem1)
        s_issue(rows2, ssem2, PHASE - 2)
        g_wait(rows3, gsem3)
        s_wait(rows2, ssem2)
        s_issue(rows3, ssem3, PHASE - 1)
        s_wait(rows3, ssem3)

    @pl.when(c == FAST_CORE)
    def _():
        for p in range(GF // PHASE):
            phase(s * PAIRC + p * PHASE)

    @pl.when(c != FAST_CORE)
    def _():
        for p in range(GS // PHASE):
            phase(s * PAIRC + GF + p * PHASE)

    plsc.subcore_barrier()

    pltpu.sync_copy(acc.at[pl.ds(s * OROWS, OROWS)],
                    out_hbm.at[c, pl.ds(s * OROWS, OROWS)])


@functools.cache
def _sc_agg_call():
    mesh = plsc.VectorSubcoreMesh(core_axis_name="c", subcore_axis_name="s",
                                  num_cores=NC, num_subcores=NS)
    return pl.kernel(
        _sc_agg_body,
        out_type=jax.ShapeDtypeStruct((NC, NACC, D), jnp.float32),
        mesh=mesh,
        scratch_types=[
            pltpu.VMEM((PHASE, CHUNK), jnp.int32),
            pltpu.VMEM((PHASE, CHUNK), jnp.int32),
            pltpu.VMEM((CHUNK, D), jnp.float32),
            pltpu.VMEM((CHUNK, D), jnp.float32),
            pltpu.VMEM((CHUNK, D), jnp.float32),
            pltpu.VMEM((CHUNK, D), jnp.float32),
            pltpu.VMEM_SHARED((NACC, D), jnp.float32),
            pltpu.SemaphoreType.DMA,
            pltpu.SemaphoreType.DMA,
            pltpu.SemaphoreType.DMA,
            pltpu.SemaphoreType.DMA,
            pltpu.SemaphoreType.DMA,
            pltpu.SemaphoreType.DMA,
            pltpu.SemaphoreType.DMA,
            pltpu.SemaphoreType.DMA,
        ],
    )


def _dense_body(x_ref, a_ref, wa_ref, ba_ref, wb_ref, bb_ref, g_ref, be_ref,
                out_ref):
    z = x_ref[...] + a_ref[0, :N] + a_ref[1, :N]
    t = jnp.maximum(
        jnp.dot(z, wa_ref[...], preferred_element_type=jnp.float32)
        + ba_ref[...], 0.0)
    u = (jnp.dot(t, wb_ref[...], preferred_element_type=jnp.float32)
         + bb_ref[...])
    v = jnp.maximum(u, 0.0)
    mu = jnp.mean(v, axis=0, keepdims=True)
    var = jnp.mean((v - mu) ** 2, axis=0, keepdims=True)
    out_ref[...] = (g_ref[...] * (v - mu) * lax.rsqrt(var + BN_EPS)
                    + be_ref[...])


_dense_call = pl.pallas_call(
    _dense_body,
    out_shape=jax.ShapeDtypeStruct((N, D), jnp.float32),
)


def kernel(x, edge_index, W1a, b1a, W1b, b1b, g1, be1,
           W2a, b2a, W2b, b2b, g2, be2,
           W5a, b5a, W5b, b5b, g5, be5):
    src = edge_index[0]
    dst = edge_index[1]
    pad = EPAD - E
    src_p = jnp.concatenate(
        [src, jnp.zeros((pad,), jnp.int32)]).reshape(EPAD // CHUNK, CHUNK)
    dst_p = jnp.concatenate(
        [dst, jnp.full((pad,), N, jnp.int32)]).reshape(EPAD // CHUNK, CHUNK)
    zeros = jnp.zeros((NACC, D), jnp.float32)

    W5b_p = jnp.pad(W5b, ((0, 0), (0, D - W5b.shape[1])))
    b5b_p = jnp.pad(b5b, (0, D - b5b.shape[0]))
    g5_p = jnp.pad(g5, (0, D - g5.shape[0]))
    be5_p = jnp.pad(be5, (0, D - be5.shape[0]))

    h = x
    layers = [
        (W1a, b1a, W1b, b1b, g1, be1),
        (W2a, b2a, W2b, b2b, g2, be2),
        (W5a, b5a, W5b_p, b5b_p, g5_p, be5_p),
    ]
    for Wa, ba, Wb, bb, g, be in layers:
        agg = _sc_agg_call()(h, src_p, dst_p, zeros)
        h = _dense_call(h, agg, Wa, ba.reshape(1, D), Wb, bb.reshape(1, D),
                        g.reshape(1, D), be.reshape(1, D))
    return h[:, :2]


# speedup vs baseline: 1.1589x; 1.1589x over previous
import functools

import jax
import jax.numpy as jnp
from jax import lax
from jax.experimental import pallas as pl
from jax.experimental.pallas import tpu as pltpu
from jax.experimental.pallas import tpu_sc as plsc

N = 10000
D = 128
E = 320000
BN_EPS = 1e-5

NC = 2
NS = 16
NW = NC * NS
CHUNK = 64
GPT = 160
PHASE = 40
PAIRC = NC * GPT
GF = 240
GS = PAIRC - GF
FAST_CORE = 0
EPAD = NW * GPT * CHUNK
NACC = 10240
ZROWS = NACC // NS
OROWS = NACC // NS

def _sc_agg_body(h_hbm, src_hbm, dst_hbm, zeros_hbm, out_hbm,
                 src_v, dst_v, rows0, rows1, rows2, rows3, acc,
                 gsem0, gsem1, gsem2, gsem3, ssem0, ssem1, ssem2, ssem3):
    c = lax.axis_index("c")
    s = lax.axis_index("s")

    pltpu.sync_copy(zeros_hbm.at[pl.ds(s * ZROWS, ZROWS)],
                    acc.at[pl.ds(s * ZROWS, ZROWS)])
    plsc.subcore_barrier()

    def g_issue(buf, sem, chunk):
        pltpu.async_copy(h_hbm.at[src_v.at[chunk]], buf, sem)

    def g_wait(buf, sem):
        pltpu.make_async_copy(h_hbm.at[src_v.at[0]], buf, sem).wait()

    def s_issue(buf, sem, chunk):
        pltpu.async_copy(buf, acc.at[dst_v.at[chunk]], sem, add=True)

    def s_wait(buf, sem):
        pltpu.make_async_copy(buf, acc.at[dst_v.at[0]], sem).wait()

    bufs = ((rows0, gsem0, ssem0), (rows1, gsem1, ssem1),
            (rows2, gsem2, ssem2), (rows3, gsem3, ssem3))

    def phase(off):
        pltpu.sync_copy(src_hbm.at[pl.ds(off, PHASE)], src_v)
        pltpu.sync_copy(dst_hbm.at[pl.ds(off, PHASE)], dst_v)

        g_issue(rows0, gsem0, 0)
        g_issue(rows1, gsem1, 1)
        g_issue(rows2, gsem2, 2)
        g_wait(rows0, gsem0)
        s_issue(rows0, ssem0, 0)
        g_issue(rows3, gsem3, 3)

        def body(j, carry):
            a = 4 * j
            for k in range(4):
                buf, gs, ss = bufs[(k + 1) % 4]
                pbuf, pgs, pss = bufs[k]
                g_wait(buf, gs)
                s_wait(pbuf, pss)
                s_issue(buf, ss, a + k + 1)
                g_issue(pbuf, pgs, a + k + 4)
            return carry

        lax.fori_loop(0, (PHASE - 8) // 4 + 1, body, 0)
        g_wait(rows1, gsem1)
        s_wait(rows0, ssem0)
        s_issue(rows1, ssem1, PHASE - 3)
        g_wait(rows2, gsem2)
        s_wait(rows1, ssem1)
        s_issue(rows2, ssem2, PHASE - 2)
        g_wait(rows3, gsem3)
        s_wait(rows2, ssem2)
        s_issue(rows3, ssem3, PHASE - 1)
        s_wait(rows3, ssem3)

    @pl.when(c == FAST_CORE)
    def _():
        for p in range(GF // PHASE):
            phase(s * PAIRC + p * PHASE)

    @pl.when(c != FAST_CORE)
    def _():
        for p in range(GS // PHASE):
            phase(s * PAIRC + GF + p * PHASE)

    plsc.subcore_barrier()

    pltpu.sync_copy(acc.at[pl.ds(s * OROWS, OROWS)],
                    out_hbm.at[c, pl.ds(s * OROWS, OROWS)])


@functools.cache
def _sc_agg_call():
    mesh = plsc.VectorSubcoreMesh(core_axis_name="c", subcore_axis_name="s",
                                  num_cores=NC, num_subcores=NS)
    return pl.kernel(
        _sc_agg_body,
        out_type=jax.ShapeDtypeStruct((NC, NACC, D), jnp.float32),
        mesh=mesh,
        scratch_types=[
            pltpu.VMEM((PHASE, CHUNK), jnp.int32),
            pltpu.VMEM((PHASE, CHUNK), jnp.int32),
            pltpu.VMEM((CHUNK, D), jnp.float32),
            pltpu.VMEM((CHUNK, D), jnp.float32),
            pltpu.VMEM((CHUNK, D), jnp.float32),
            pltpu.VMEM((CHUNK, D), jnp.float32),
            pltpu.VMEM_SHARED((NACC, D), jnp.float32),
            pltpu.SemaphoreType.DMA,
            pltpu.SemaphoreType.DMA,
            pltpu.SemaphoreType.DMA,
            pltpu.SemaphoreType.DMA,
            pltpu.SemaphoreType.DMA,
            pltpu.SemaphoreType.DMA,
            pltpu.SemaphoreType.DMA,
            pltpu.SemaphoreType.DMA,
        ],
    )


def _dense_body(x_ref, a_ref, wa_ref, ba_ref, wb_ref, bb_ref, g_ref, be_ref,
                out_ref):
    z = x_ref[...] + a_ref[0, :N] + a_ref[1, :N]
    t = jnp.maximum(
        jnp.dot(z, wa_ref[...], preferred_element_type=jnp.float32)
        + ba_ref[...], 0.0)
    u = (jnp.dot(t, wb_ref[...], preferred_element_type=jnp.float32)
         + bb_ref[...])
    v = jnp.maximum(u, 0.0)
    mu = jnp.mean(v, axis=0, keepdims=True)
    var = jnp.mean((v - mu) ** 2, axis=0, keepdims=True)
    out_ref[...] = (g_ref[...] * (v - mu) * lax.rsqrt(var + BN_EPS)
                    + be_ref[...])


_dense_call = pl.pallas_call(
    _dense_body,
    out_shape=jax.ShapeDtypeStruct((N, D), jnp.float32),
)


def kernel(x, edge_index, W1a, b1a, W1b, b1b, g1, be1,
           W2a, b2a, W2b, b2b, g2, be2,
           W5a, b5a, W5b, b5b, g5, be5):
    src = edge_index[0]
    dst = edge_index[1]
    pad = EPAD - E
    src_p = jnp.concatenate(
        [src, jnp.zeros((pad,), jnp.int32)]).reshape(EPAD // CHUNK, CHUNK)
    dst_p = jnp.concatenate(
        [dst, jnp.full((pad,), N, jnp.int32)]).reshape(EPAD // CHUNK, CHUNK)
    zeros = jnp.zeros((NACC, D), jnp.float32)

    W5b_p = jnp.pad(W5b, ((0, 0), (0, D - W5b.shape[1])))
    b5b_p = jnp.pad(b5b, (0, D - b5b.shape[0]))
    g5_p = jnp.pad(g5, (0, D - g5.shape[0]))
    be5_p = jnp.pad(be5, (0, D - be5.shape[0]))

    h = x
    layers = [
        (W1a, b1a, W1b, b1b, g1, be1),
        (W2a, b2a, W2b, b2b, g2, be2),
        (W5a, b5a, W5b_p, b5b_p, g5_p, be5_p),
    ]
    for Wa, ba, Wb, bb, g, be in layers:
        agg = _sc_agg_call()(h, src_p, dst_p, zeros)
        h = _dense_call(h, agg, Wa, ba.reshape(1, D), Wb, bb.reshape(1, D),
                        g.reshape(1, D), be.reshape(1, D))
    return h[:, :2]
